# baseline (speedup 1.0000x reference)
import functools

import jax
import jax.numpy as jnp
from jax import lax
from jax.experimental import pallas as pl
from jax.experimental.pallas import tpu as pltpu
from jax.experimental.pallas import tpu_sc as plsc

_FIELD_DIM = 100000
_NUM_F = 26
_EMBED_D = 16
_BATCH = 16384

_NW = 32
_SAMPLES_PER_W = _BATCH // _NW
_CHUNK = 128
_NCHUNK = _SAMPLES_PER_W // _CHUNK
_G = 16
_NGROUP = _CHUNK // _G
_GELEM = _G * _NUM_F
_DMA_PIECES = [(0, 128), (128, 128), (256, 128), (384, 32)]


def _fm_kernel(xt_hbm, wemb_hbm, wlin_hbm, bias_hbm, out_hbm,
               xst_v, slots_v, xis_v, cols_v, eb_v, lin_v, outb_v, bias_v,
               sem_e, sem_l):
    wid = lax.axis_index("s") * 2 + lax.axis_index("c")

    pltpu.sync_copy(bias_hbm, bias_v)
    bias_vec = bias_v[pl.ds(0, 16)]
    iota = lax.iota(jnp.int32, 16)

    def chunk_body(k, carry):
        c0 = wid * _SAMPLES_PER_W + k * _CHUNK
        pltpu.sync_copy(xt_hbm.at[:, pl.ds(c0, _CHUNK)], xst_v)

        def group_body(g, carry2):
            for f in range(_NUM_F):
                xi = xst_v[f, pl.ds(g * 16, 16)] + (f * _FIELD_DIM - 1)
                slots_v[pl.ds(f * 16, 16)] = lax.shift_right_arithmetic(xi, 3)
                cols_v[pl.ds(f * 16, 16)] = lax.shift_left(
                    jnp.bitwise_and(xi, 7), 4)
                xis_v[pl.ds(f * 16, 16)] = xi

            handles = []
            for (o, n) in _DMA_PIECES:
                piece = pl.ds(o, n)
                handles.append(pltpu.async_copy(
                    wemb_hbm.at[slots_v.at[piece]], eb_v.at[piece], sem_e))
                handles.append(pltpu.async_copy(
                    wlin_hbm.at[xis_v.at[piece]], lin_v.at[piece], sem_l))
            for h in handles:
                h.wait()

            s = [jnp.zeros((16,), jnp.float32) for _ in range(_EMBED_D)]
            sq = [jnp.zeros((16,), jnp.float32) for _ in range(_EMBED_D)]
            for f in range(_NUM_F):
                rowv = iota + f * 16
                colb = cols_v[pl.ds(f * 16, 16)]
                for d in range(_EMBED_D):
                    gv = plsc.load_gather(eb_v, [rowv, colb + d])
                    s[d] = s[d] + gv
                    sq[d] = sq[d] + gv * gv
            acc = jnp.zeros((16,), jnp.float32)
            for d in range(_EMBED_D):
                acc = acc + (s[d] * s[d] - sq[d])

            lacc = jnp.zeros((16,), jnp.float32)
            for f in range(_NUM_F):
                lacc = lacc + lin_v[pl.ds(f * 16, 16)]

            z = lacc + bias_vec + 0.5 * acc
            outb_v[pl.ds(g * 16, 16)] = 1.0 / (1.0 + jnp.exp(-z))
            return carry2

        lax.fori_loop(0, _NGROUP, group_body, 0)
        pltpu.sync_copy(outb_v, out_hbm.at[pl.ds(c0, _CHUNK)])
        return carry

    lax.fori_loop(0, _NCHUNK, chunk_body, 0)


def kernel(x, W_emb, W_lin, bias):
    xt = x.astype(jnp.int32).T
    wemb128 = W_emb.reshape(-1, 128)
    wlin1d = W_lin.reshape(-1)

    mesh = plsc.VectorSubcoreMesh(core_axis_name="c", subcore_axis_name="s")
    run = functools.partial(
        pl.kernel,
        mesh=mesh,
        compiler_params=pltpu.CompilerParams(needs_layout_passes=False),
        out_type=jax.ShapeDtypeStruct((_BATCH,), jnp.float32),
        scratch_types=[
            pltpu.VMEM((_NUM_F, _CHUNK), jnp.int32),
            pltpu.VMEM((_GELEM,), jnp.int32),
            pltpu.VMEM((_GELEM,), jnp.int32),
            pltpu.VMEM((_GELEM,), jnp.int32),
            pltpu.VMEM((_GELEM, 128), jnp.float32),
            pltpu.VMEM((_GELEM,), jnp.float32),
            pltpu.VMEM((_CHUNK,), jnp.float32),
            pltpu.VMEM((16,), jnp.float32),
            pltpu.SemaphoreType.DMA,
            pltpu.SemaphoreType.DMA,
        ],
    )(_fm_kernel)
    return run(xt, wemb128, wlin1d, jnp.broadcast_to(bias, (16,)))
